# Initial kernel scaffold; baseline (speedup 1.0000x reference)
import jax
import jax.numpy as jnp
from jax.experimental import pallas as pl


def kernel(x, enc_embed_w, enc_embed_b, enc_norm_g, enc_norm_b, enc_dw_w, enc_dw_b, enc_ln_g, enc_ln_b, enc_w1, enc_b1, enc_w2, enc_b2, enc_gamma, enc_fnorm_g, enc_fnorm_b, enc_out_w, enc_out_b, vq_in_w, vq_in_b, codebook, vq_out_w, vq_out_b):
    raise NotImplementedError("write your pallas kernel here")


# speedup vs baseline: 4.1995x; 4.1995x over previous
import jax
import jax.numpy as jnp
from jax.experimental import pallas as pl
from jax.experimental.pallas import tpu as pltpu

B, T, H, D, I, L, CB, CD = 8, 2048, 1024, 384, 2048, 12, 8192, 8
_EPS = 1e-6


def _dot(a, b):
    return jnp.dot(a.astype(jnp.bfloat16), b.astype(jnp.bfloat16),
                   preferred_element_type=jnp.float32)


def _ln_chunk(h, g, b):
    mu = jnp.mean(h, axis=-1, keepdims=True)
    d = h - mu
    var = jnp.mean(d * d, axis=-1, keepdims=True)
    return d / jnp.sqrt(var + _EPS) * g + b


_TC1 = 512


def _shifted_rows(ref, o, n, w):
    if o >= 0 and o + n <= T:
        return ref[0, o:o + n, :]
    if o < 0:
        return jnp.concatenate(
            [jnp.zeros((-o, w), jnp.float32), ref[0, 0:o + n, :]], axis=0)
    return jnp.concatenate(
        [ref[0, o:T, :], jnp.zeros((o + n - T, w), jnp.float32)], axis=0)


def _embed_body(x_ref, ew_ref, eb_ref, ng_ref, nb_ref, h0_ref):
    for tc in range(T // _TC1):
        r = tc * _TC1
        acc = _dot(_shifted_rows(x_ref, r - 3, _TC1, H), ew_ref[0])
        for k in range(1, 7):
            acc = acc + _dot(_shifted_rows(x_ref, r + k - 3, _TC1, H), ew_ref[k])
        hh = acc + eb_ref[...]
        h0_ref[0, r:r + _TC1, :] = _ln_chunk(hh, ng_ref[...], nb_ref[...])


_TC2 = 256


def _block_body(h0_ref, dw_ref, dwb_ref, lng_ref, lnb_ref, w1_ref, b1_ref,
                w2_ref, b2_ref, gm_ref, out_ref, hpad_ref):
    l = pl.program_id(1)

    @pl.when(l == 0)
    def _():
        hpad_ref[0:3, :] = jnp.zeros((3, D), jnp.float32)
        hpad_ref[T + 3:T + 8, :] = jnp.zeros((5, D), jnp.float32)
        for tc in range(T // _TC2):
            r = tc * _TC2
            out_ref[0, r:r + _TC2, :] = h0_ref[0, r:r + _TC2, :]

    for tc in range(T // _TC2):
        r = tc * _TC2
        hpad_ref[3 + r:3 + r + _TC2, :] = out_ref[0, r:r + _TC2, :]

    for tc in range(T // _TC2):
        r = tc * _TC2
        y = hpad_ref[r:r + _TC2, :] * dw_ref[0, 0:1, :]
        for k in range(1, 7):
            y = y + hpad_ref[k + r:k + r + _TC2, :] * dw_ref[0, k:k + 1, :]
        y = y + dwb_ref[0]
        y = _ln_chunk(y, lng_ref[0], lnb_ref[0])
        z = _dot(y, w1_ref[0]) + b1_ref[0]
        z = 0.5 * z * (1.0 + jax.lax.erf(z * 0.7071067811865476))
        z = _dot(z, w2_ref[0]) + b2_ref[0]
        out_ref[0, r:r + _TC2, :] = out_ref[0, r:r + _TC2, :] + z * gm_ref[0]


_TC3 = 256


def _final_body(h_ref, fg_ref, fb_ref, ow_ref, ob_ref, viw_ref, vib_ref,
                cbt_ref, vow_ref, vob_ref, idx_ref, q_ref):
    cb = cbt_ref[...]
    ss = jnp.sum(cb * cb, axis=0, keepdims=True)
    cbn = (cb / jnp.maximum(jnp.sqrt(ss), 1e-12)).astype(jnp.bfloat16)
    cb16 = cb.astype(jnp.bfloat16)

    for tc in range(T // _TC3):
        r = tc * _TC3
        hh = h_ref[0, r:r + _TC3, :]
        hn = _ln_chunk(hh, fg_ref[...], fb_ref[...])
        e = _dot(hn, ow_ref[...]) + ob_ref[...]
        z = _dot(e, viw_ref[...]) + vib_ref[...]
        zn = z / jnp.maximum(jnp.sqrt(jnp.sum(z * z, axis=-1, keepdims=True)),
                             1e-12)
        sim = jnp.dot(zn.astype(jnp.bfloat16), cbn,
                      preferred_element_type=jnp.float32)
        ii = jnp.argmax(sim, axis=-1)
        idx_ref[0, :, r:r + _TC3] = ii.reshape(1, _TC3)
        oh = (jax.lax.broadcasted_iota(jnp.int32, (_TC3, CB), 1)
              == ii[:, None]).astype(jnp.bfloat16)
        zq = jax.lax.dot_general(oh, cb16, (((1,), (1,)), ((), ())),
                                 preferred_element_type=jnp.float32)
        q_ref[0, r:r + _TC3, :] = _dot(zq, vow_ref[...]) + vob_ref[...]


def kernel(x, enc_embed_w, enc_embed_b, enc_norm_g, enc_norm_b, enc_dw_w,
           enc_dw_b, enc_ln_g, enc_ln_b, enc_w1, enc_b1, enc_w2, enc_b2,
           enc_gamma, enc_fnorm_g, enc_fnorm_b, enc_out_w, enc_out_b,
           vq_in_w, vq_in_b, codebook, vq_out_w, vq_out_b):
    f32 = jnp.float32
    bf16 = jnp.bfloat16
    ew_t = enc_embed_w.transpose(2, 1, 0).astype(bf16)
    dw_t = enc_dw_w.reshape(L, D, 7).transpose(0, 2, 1)
    w1_t = enc_w1.transpose(0, 2, 1).astype(bf16)
    w2_t = enc_w2.transpose(0, 2, 1).astype(bf16)
    ow_t = enc_out_w.T.astype(bf16)
    viw_t = vq_in_w.T.astype(bf16)
    vow_t = vq_out_w.T.astype(bf16)
    cb_t = codebook.T
    r1 = lambda v: v.reshape(1, -1)
    r3 = lambda v: v.reshape(v.shape[0], 1, v.shape[1])

    params = pltpu.CompilerParams(
        dimension_semantics=("parallel", "arbitrary"),
        vmem_limit_bytes=56 * 1024 * 1024,
    )
    params1 = pltpu.CompilerParams(
        dimension_semantics=("parallel",),
        vmem_limit_bytes=56 * 1024 * 1024,
    )

    h0 = pl.pallas_call(
        _embed_body,
        grid=(B,),
        in_specs=[
            pl.BlockSpec((1, T, H), lambda b: (b, 0, 0)),
            pl.BlockSpec((7, H, D), lambda b: (0, 0, 0)),
            pl.BlockSpec((1, D), lambda b: (0, 0)),
            pl.BlockSpec((1, D), lambda b: (0, 0)),
            pl.BlockSpec((1, D), lambda b: (0, 0)),
        ],
        out_specs=pl.BlockSpec((1, T, D), lambda b: (b, 0, 0)),
        out_shape=jax.ShapeDtypeStruct((B, T, D), f32),
        compiler_params=params1,
        name="rc_embed",
    )(x, ew_t, r1(enc_embed_b), r1(enc_norm_g), r1(enc_norm_b))

    hL = pl.pallas_call(
        _block_body,
        grid=(B, L),
        in_specs=[
            pl.BlockSpec((1, T, D), lambda b, l: (b, 0, 0)),
            pl.BlockSpec((1, 7, D), lambda b, l: (l, 0, 0)),
            pl.BlockSpec((1, 1, D), lambda b, l: (l, 0, 0)),
            pl.BlockSpec((1, 1, D), lambda b, l: (l, 0, 0)),
            pl.BlockSpec((1, 1, D), lambda b, l: (l, 0, 0)),
            pl.BlockSpec((1, D, I), lambda b, l: (l, 0, 0)),
            pl.BlockSpec((1, 1, I), lambda b, l: (l, 0, 0)),
            pl.BlockSpec((1, I, D), lambda b, l: (l, 0, 0)),
            pl.BlockSpec((1, 1, D), lambda b, l: (l, 0, 0)),
            pl.BlockSpec((1, 1, D), lambda b, l: (l, 0, 0)),
        ],
        out_specs=pl.BlockSpec((1, T, D), lambda b, l: (b, 0, 0)),
        out_shape=jax.ShapeDtypeStruct((B, T, D), f32),
        scratch_shapes=[pltpu.VMEM((T + 8, D), f32)],
        compiler_params=params,
        name="rc_blocks",
    )(h0, dw_t, r3(enc_dw_b), r3(enc_ln_g), r3(enc_ln_b), w1_t, r3(enc_b1),
      w2_t, r3(enc_b2), r3(enc_gamma))

    idx3, quant = pl.pallas_call(
        _final_body,
        grid=(B,),
        in_specs=[
            pl.BlockSpec((1, T, D), lambda b: (b, 0, 0)),
            pl.BlockSpec((1, D), lambda b: (0, 0)),
            pl.BlockSpec((1, D), lambda b: (0, 0)),
            pl.BlockSpec((D, H), lambda b: (0, 0)),
            pl.BlockSpec((1, H), lambda b: (0, 0)),
            pl.BlockSpec((H, CD), lambda b: (0, 0)),
            pl.BlockSpec((1, CD), lambda b: (0, 0)),
            pl.BlockSpec((CD, CB), lambda b: (0, 0)),
            pl.BlockSpec((CD, H), lambda b: (0, 0)),
            pl.BlockSpec((1, H), lambda b: (0, 0)),
        ],
        out_specs=[
            pl.BlockSpec((1, 1, T), lambda b: (b, 0, 0)),
            pl.BlockSpec((1, T, H), lambda b: (b, 0, 0)),
        ],
        out_shape=[
            jax.ShapeDtypeStruct((B, 1, T), jnp.int32),
            jax.ShapeDtypeStruct((B, T, H), f32),
        ],
        compiler_params=params1,
        name="rc_final_vq",
    )(hL, r1(enc_fnorm_g), r1(enc_fnorm_b), ow_t, r1(enc_out_b), viw_t,
      r1(vq_in_b), cb_t, vow_t, r1(vq_out_b))

    return idx3.reshape(B, T), quant
